# baseline (speedup 1.0000x reference)
import functools

import jax
import jax.numpy as jnp
from jax import lax
from jax.experimental import pallas as pl
from jax.experimental.pallas import tpu as pltpu
from jax.experimental.pallas import tpu_sc as plsc

D_MODEL = 64
BATCH = 1024
SEQ_LEN = 200
NC, NS, LANES = 2, 16, 16
NW = NC * NS
RPW = BATCH // NW
NBUF = 4


def _pos_encoding(seq_len, d_model):
    i_model = jnp.repeat(jnp.arange(d_model // 2), 2)
    div_term = jnp.exp(
        i_model.astype(jnp.float32) / d_model * jnp.log(jnp.float32(10000.0))
    )
    pos = jnp.arange(seq_len, dtype=jnp.float32)[:, None] / div_term
    even = (jnp.arange(d_model) % 2) == 0
    return jnp.where(even[None, :], jnp.sin(pos), jnp.cos(pos))


def _body(x_ref, tab_ref, pe_ref, out_ref, idxq_v, idx_v, bufs, pe_v, sems):
    gsems, osems = sems[:NBUF], sems[NBUF:]
    wid = lax.axis_index("s") * NC + lax.axis_index("c")
    b0 = wid * RPW
    j = b0 // 128
    c0 = b0 % 128
    pltpu.sync_copy(x_ref.at[:, j, :, pl.ds(c0, RPW)], idxq_v.at[pl.ds(0, 25)])
    pltpu.sync_copy(pe_ref, pe_v)

    iota = lax.iota(jnp.int32, LANES)

    @pl.loop(0, RPW)
    def _c(c):
        cv = jnp.broadcast_to(c, (LANES,))
        for m in range(13):
            lv = iota + m * LANES
            vals = plsc.load_gather(idxq_v, [lv >> 3, lv & 7, cv])
            idx_v[c, pl.ds(m * LANES, LANES)] = vals

    @pl.loop(0, RPW // NBUF)
    def _group(t):
        r0 = t * NBUF
        for k in range(NBUF):
            buf = bufs.at[k]

            @pl.loop(0, SEQ_LEN, unroll=8)
            def _fill(r):
                for q in range(D_MODEL // LANES):
                    sl = pl.ds(q * LANES, LANES)
                    buf[r, sl] = pe_v[r, sl]

        gds = [
            pltpu.async_copy(
                tab_ref.at[idx_v.at[r0 + k, pl.ds(0, SEQ_LEN)]],
                bufs.at[k],
                gsems[k],
                add=True,
            )
            for k in range(NBUF)
        ]
        ods = []
        for k in range(NBUF):
            gds[k].wait()
            buf = bufs.at[k]
            ods.append(
                pltpu.async_copy(
                    buf.at[:, pl.ds(0, D_MODEL)],
                    out_ref.at[b0 + r0 + k, :, pl.ds(0, D_MODEL)],
                    osems[k],
                )
            )
        for d in ods:
            d.wait()


@functools.partial(jax.jit, static_argnums=())
def _emb_lookup(x, emb_weight, pe):
    mesh = plsc.VectorSubcoreMesh(
        core_axis_name="c", subcore_axis_name="s", num_cores=NC, num_subcores=NS
    )
    f = pl.kernel(
        _body,
        out_type=jax.ShapeDtypeStruct((BATCH, SEQ_LEN, 2 * D_MODEL), jnp.float32),
        mesh=mesh,
        scratch_types=[
            pltpu.VMEM((26, 8, RPW), jnp.int32),
            pltpu.VMEM((RPW, 208), jnp.int32),
            pltpu.VMEM((NBUF, SEQ_LEN, 2 * D_MODEL), jnp.float32),
            pltpu.VMEM((SEQ_LEN, D_MODEL), jnp.float32),
            [pltpu.SemaphoreType.DMA] * (2 * NBUF),
        ],
        compiler_params=pltpu.CompilerParams(use_tc_tiling_on_sc=False, needs_layout_passes=False),
    )
    return f(x, emb_weight, pe)


def kernel(x, emb_weight):
    pe = _pos_encoding(SEQ_LEN, D_MODEL)
    xq = x.T.reshape(25, 8, 8, 128).transpose(0, 2, 1, 3)
    tab128 = jnp.pad(emb_weight, ((0, 0), (0, D_MODEL)))
    out128 = _emb_lookup(xq, tab128, pe)
    return out128[:, :, :D_MODEL]


# speedup vs baseline: 1.0147x; 1.0147x over previous
import functools

import jax
import jax.numpy as jnp
from jax import lax
from jax.experimental import pallas as pl
from jax.experimental.pallas import tpu as pltpu
from jax.experimental.pallas import tpu_sc as plsc

D_MODEL = 64
BATCH = 1024
SEQ_LEN = 200
NC, NS, LANES = 2, 16, 16
NW = NC * NS
RPW = BATCH // NW
NBUF = 4


def _pos_encoding(seq_len, d_model):
    i_model = jnp.repeat(jnp.arange(d_model // 2), 2)
    div_term = jnp.exp(
        i_model.astype(jnp.float32) / d_model * jnp.log(jnp.float32(10000.0))
    )
    pos = jnp.arange(seq_len, dtype=jnp.float32)[:, None] / div_term
    even = (jnp.arange(d_model) % 2) == 0
    return jnp.where(even[None, :], jnp.sin(pos), jnp.cos(pos))


def _body(x_ref, tab_ref, pe_ref, out_ref, idxq_v, idx_v, bufs, pe_v, sems):
    gsems, osems = sems[:NBUF], sems[NBUF:]
    wid = lax.axis_index("s") * NC + lax.axis_index("c")
    b0 = wid * RPW
    j = b0 // 128
    c0 = b0 % 128
    pltpu.sync_copy(x_ref.at[:, j, :, pl.ds(c0, RPW)], idxq_v.at[pl.ds(0, 25)])
    pltpu.sync_copy(pe_ref, pe_v)

    iota = lax.iota(jnp.int32, LANES)

    @pl.loop(0, RPW)
    def _c(c):
        cv = jnp.broadcast_to(c, (LANES,))
        for m in range(13):
            lv = iota + m * LANES
            vals = plsc.load_gather(idxq_v, [lv >> 3, lv & 7, cv])
            idx_v[c, pl.ds(m * LANES, LANES)] = vals

    @pl.loop(0, RPW // NBUF)
    def _group(t):
        r0 = t * NBUF
        for k in range(NBUF):
            buf = bufs.at[k]

            @pl.loop(0, SEQ_LEN, unroll=8)
            def _fill(r):
                for q in range(D_MODEL // LANES):
                    sl = pl.ds(q * LANES, LANES)
                    buf[r, sl] = pe_v[r, sl]

        gds = [
            pltpu.async_copy(
                tab_ref.at[idx_v.at[r0 + k, pl.ds(0, SEQ_LEN)]],
                bufs.at[k],
                gsems[k],
                add=True,
            )
            for k in range(NBUF)
        ]
        ods = []
        for k in range(NBUF):
            gds[k].wait()
            buf = bufs.at[k]
            ods.append(
                pltpu.async_copy(
                    buf, out_ref.at[b0 + r0 + k, :, pl.ds(0, D_MODEL)], osems[k]
                )
            )
        for d in ods:
            d.wait()


@functools.partial(jax.jit, static_argnums=())
def _emb_lookup(x, emb_weight, pe):
    mesh = plsc.VectorSubcoreMesh(
        core_axis_name="c", subcore_axis_name="s", num_cores=NC, num_subcores=NS
    )
    f = pl.kernel(
        _body,
        out_type=jax.ShapeDtypeStruct((BATCH, SEQ_LEN, 2 * D_MODEL), jnp.float32),
        mesh=mesh,
        scratch_types=[
            pltpu.VMEM((26, 8, RPW), jnp.int32),
            pltpu.VMEM((RPW, 208), jnp.int32),
            pltpu.VMEM((NBUF, SEQ_LEN, D_MODEL), jnp.float32),
            pltpu.VMEM((SEQ_LEN, D_MODEL), jnp.float32),
            [pltpu.SemaphoreType.DMA] * (2 * NBUF),
        ],
        compiler_params=pltpu.CompilerParams(use_tc_tiling_on_sc=False, needs_layout_passes=False),
    )
    return f(x, emb_weight, pe)


def kernel(x, emb_weight):
    pe = _pos_encoding(SEQ_LEN, D_MODEL)
    xq = x.T.reshape(25, 8, 8, 128).transpose(0, 2, 1, 3)
    out128 = _emb_lookup(xq, emb_weight, pe)
    return out128[:, :, :D_MODEL]


# speedup vs baseline: 1.0456x; 1.0305x over previous
import functools

import jax
import jax.numpy as jnp
from jax import lax
from jax.experimental import pallas as pl
from jax.experimental.pallas import tpu as pltpu
from jax.experimental.pallas import tpu_sc as plsc

D_MODEL = 64
BATCH = 1024
SEQ_LEN = 200
NC, NS, LANES = 2, 16, 16
NW = NC * NS
RPW = BATCH // NW
NBUF = 8


def _pos_encoding(seq_len, d_model):
    i_model = jnp.repeat(jnp.arange(d_model // 2), 2)
    div_term = jnp.exp(
        i_model.astype(jnp.float32) / d_model * jnp.log(jnp.float32(10000.0))
    )
    pos = jnp.arange(seq_len, dtype=jnp.float32)[:, None] / div_term
    even = (jnp.arange(d_model) % 2) == 0
    return jnp.where(even[None, :], jnp.sin(pos), jnp.cos(pos))


def _body(x_ref, tab_ref, pe_ref, out_ref, idxq_v, idx_v, bufs, pe_v, sems):
    gsems, osems = sems[:NBUF], sems[NBUF:]
    wid = lax.axis_index("s") * NC + lax.axis_index("c")
    b0 = wid * RPW
    j = b0 // 128
    c0 = b0 % 128
    pltpu.sync_copy(x_ref.at[:, j, :, pl.ds(c0, RPW)], idxq_v.at[pl.ds(0, 25)])
    pltpu.sync_copy(pe_ref, pe_v)

    iota = lax.iota(jnp.int32, LANES)

    @pl.loop(0, RPW)
    def _c(c):
        cv = jnp.broadcast_to(c, (LANES,))
        for m in range(13):
            lv = iota + m * LANES
            vals = plsc.load_gather(idxq_v, [lv >> 3, lv & 7, cv])
            idx_v[c, pl.ds(m * LANES, LANES)] = vals

    @pl.loop(0, RPW // NBUF)
    def _group(t):
        r0 = t * NBUF
        for k in range(NBUF):
            buf = bufs.at[k]

            @pl.loop(0, SEQ_LEN, unroll=8)
            def _fill(r):
                for q in range(D_MODEL // LANES):
                    sl = pl.ds(q * LANES, LANES)
                    buf[r, sl] = pe_v[r, sl]

        gds = [
            pltpu.async_copy(
                tab_ref.at[idx_v.at[r0 + k, pl.ds(0, SEQ_LEN)]],
                bufs.at[k],
                gsems[k],
                add=True,
            )
            for k in range(NBUF)
        ]
        ods = []
        for k in range(NBUF):
            gds[k].wait()
            buf = bufs.at[k]
            ods.append(
                pltpu.async_copy(
                    buf, out_ref.at[b0 + r0 + k, :, pl.ds(0, D_MODEL)], osems[k]
                )
            )
        for d in ods:
            d.wait()


@functools.partial(jax.jit, static_argnums=())
def _emb_lookup(x, emb_weight, pe):
    mesh = plsc.VectorSubcoreMesh(
        core_axis_name="c", subcore_axis_name="s", num_cores=NC, num_subcores=NS
    )
    f = pl.kernel(
        _body,
        out_type=jax.ShapeDtypeStruct((BATCH, SEQ_LEN, 2 * D_MODEL), jnp.float32),
        mesh=mesh,
        scratch_types=[
            pltpu.VMEM((26, 8, RPW), jnp.int32),
            pltpu.VMEM((RPW, 208), jnp.int32),
            pltpu.VMEM((NBUF, SEQ_LEN, D_MODEL), jnp.float32),
            pltpu.VMEM((SEQ_LEN, D_MODEL), jnp.float32),
            [pltpu.SemaphoreType.DMA] * (2 * NBUF),
        ],
        compiler_params=pltpu.CompilerParams(use_tc_tiling_on_sc=False, needs_layout_passes=False),
    )
    return f(x, emb_weight, pe)


def kernel(x, emb_weight):
    pe = _pos_encoding(SEQ_LEN, D_MODEL)
    xq = x.T.reshape(25, 8, 8, 128).transpose(0, 2, 1, 3)
    out128 = _emb_lookup(xq, emb_weight, pe)
    return out128[:, :, :D_MODEL]
